# Initial kernel scaffold; baseline (speedup 1.0000x reference)
import functools

import jax
import jax.numpy as jnp
from jax import lax
from jax.experimental import pallas as pl
from jax.experimental.pallas import tpu as pltpu
from jax.experimental.pallas import tpu_sc as plsc

N = 10000
E = 320000
N_PAD = 10240
NC = 2
NS = 16
NW = NC * NS
EPW = E // NW
CH = N_PAD // NS

_sc_mesh = plsc.VectorSubcoreMesh(core_axis_name="c", subcore_axis_name="s")


@functools.partial(
    pl.kernel,
    out_type=jax.ShapeDtypeStruct((NC, N_PAD), jnp.float32),
    mesh=_sc_mesh,
    scratch_types=[
        pltpu.VMEM((EPW,), jnp.int32),
        pltpu.VMEM((EPW,), jnp.float32),
        pltpu.VMEM_SHARED((N_PAD,), jnp.float32),
    ],
)
def _deg_kernel(dst_hbm, ones_hbm, zeros_hbm, out_hbm, idx_v, ones_v, deg_sh):
    c = lax.axis_index("c")
    s = lax.axis_index("s")
    wid = s * NC + c
    pltpu.sync_copy(zeros_hbm.at[pl.ds(s * CH, CH)], deg_sh.at[pl.ds(s * CH, CH)])
    pltpu.sync_copy(dst_hbm.at[pl.ds(wid * EPW, EPW)], idx_v)
    pltpu.sync_copy(ones_hbm, ones_v)
    plsc.subcore_barrier()
    pltpu.sync_copy(ones_v, deg_sh.at[idx_v], add=True)
    plsc.subcore_barrier()
    pltpu.sync_copy(deg_sh.at[pl.ds(s * CH, CH)], out_hbm.at[c, pl.ds(s * CH, CH)])


def kernel(in_feat, edge_index, W1, b1, W2, b2):
    src = edge_index[0]
    dst = edge_index[1]
    ones_e = jnp.ones((EPW,), jnp.float32)
    zeros_n = jnp.zeros((N_PAD,), jnp.float32)
    degp = _deg_kernel(dst, ones_e, zeros_n)
    deg = degp[0, :N] + degp[1, :N] + 1.0
    dis = lax.rsqrt(deg)

    y1 = (in_feat @ W1) * dis[:, None]
    acc1 = jnp.zeros_like(y1).at[dst].add(y1[src])
    h = jax.nn.relu(dis[:, None] * (acc1 + y1) + b1[None, :])
    y2 = (h @ W2) * dis[:, None]
    acc2 = jnp.zeros_like(y2).at[dst].add(y2[src])
    out = dis[:, None] * (acc2 + y2) + b2[None, :]
    return out


# speedup vs baseline: 9.8965x; 9.8965x over previous
import functools

import jax
import jax.numpy as jnp
from jax import lax
from jax.experimental import pallas as pl
from jax.experimental.pallas import tpu as pltpu
from jax.experimental.pallas import tpu_sc as plsc

N = 10000
E = 320000
N_PAD = 10240
NC = 2
NS = 16
NW = NC * NS
EPW = E // NW
CH = N_PAD // NS

_sc_mesh = plsc.VectorSubcoreMesh(core_axis_name="c", subcore_axis_name="s")


@functools.partial(
    pl.kernel,
    out_type=jax.ShapeDtypeStruct((NC, N_PAD), jnp.float32),
    mesh=_sc_mesh,
    scratch_types=[
        pltpu.VMEM((EPW,), jnp.int32),
        pltpu.VMEM((EPW,), jnp.float32),
        pltpu.VMEM_SHARED((N_PAD,), jnp.float32),
    ],
)
def _deg_kernel(dst_hbm, ones_hbm, zeros_hbm, out_hbm, idx_v, ones_v, deg_sh):
    c = lax.axis_index("c")
    s = lax.axis_index("s")
    wid = s * NC + c
    pltpu.sync_copy(zeros_hbm.at[pl.ds(s * CH, CH)], deg_sh.at[pl.ds(s * CH, CH)])
    pltpu.sync_copy(dst_hbm.at[pl.ds(wid * EPW, EPW)], idx_v)
    pltpu.sync_copy(ones_hbm, ones_v)
    plsc.subcore_barrier()
    pltpu.sync_copy(ones_v, deg_sh.at[idx_v], add=True)
    plsc.subcore_barrier()
    pltpu.sync_copy(deg_sh.at[pl.ds(s * CH, CH)], out_hbm.at[c, pl.ds(s * CH, CH)])


B = 128
EPT = 10240
NB = EPT // B
E_PAD = NW * EPT
DUMMY_DST = N_PAD - 8


def _make_agg_kernel(d):
    @functools.partial(
        pl.kernel,
        out_type=jax.ShapeDtypeStruct((NC, N_PAD, d), jnp.float32),
        mesh=_sc_mesh,
        scratch_types=[
            pltpu.VMEM((NB, B), jnp.int32),
            pltpu.VMEM((NB, B), jnp.int32),
            pltpu.VMEM((B, d), jnp.float32),
            pltpu.VMEM_SHARED((N_PAD, d), jnp.float32),
            pltpu.SemaphoreType.DMA,
        ],
    )
    def _agg(y_hbm, src_hbm, dst_hbm, zeros_hbm, out_hbm,
             src_v, dst_v, rows_v, acc_sh, sem):
        c = lax.axis_index("c")
        s = lax.axis_index("s")
        wid = s * NC + c
        pltpu.sync_copy(zeros_hbm.at[pl.ds(s * CH, CH), :],
                        acc_sh.at[pl.ds(s * CH, CH), :])
        pltpu.sync_copy(src_hbm.at[wid], src_v)
        pltpu.sync_copy(dst_hbm.at[wid], dst_v)
        plsc.subcore_barrier()

        def body(b, carry):
            pltpu.async_copy(y_hbm.at[src_v.at[b]], rows_v, sem).wait()
            pltpu.sync_copy(rows_v, acc_sh.at[dst_v.at[b]], add=True)
            return carry

        lax.fori_loop(0, NB, body, 0)
        plsc.subcore_barrier()
        pltpu.sync_copy(acc_sh.at[pl.ds(s * CH, CH), :],
                        out_hbm.at[c, pl.ds(s * CH, CH), :])

    return _agg


_agg_128 = _make_agg_kernel(128)


def kernel(in_feat, edge_index, W1, b1, W2, b2):
    src = edge_index[0]
    dst = edge_index[1]
    ones_e = jnp.ones((EPW,), jnp.float32)
    zeros_n = jnp.zeros((N_PAD,), jnp.float32)
    degp = _deg_kernel(dst, ones_e, zeros_n)
    deg = degp[0, :N] + degp[1, :N] + 1.0
    dis = lax.rsqrt(deg)

    epad = jnp.zeros((E_PAD - E,), jnp.int32)
    src3 = jnp.concatenate([src, epad]).reshape(NW, NB, B)
    dst3 = jnp.concatenate([dst, epad + DUMMY_DST]).reshape(NW, NB, B)
    zeros_128 = jnp.zeros((N_PAD, 128), jnp.float32)
    W2p = jnp.zeros((128, 128), jnp.float32).at[:, :40].set(W2)
    b2p = jnp.zeros((128,), jnp.float32).at[:40].set(b2)

    y1 = (in_feat @ W1) * dis[:, None]
    y1p = jnp.zeros((N_PAD, 128), jnp.float32).at[:N].set(y1)
    a1 = _agg_128(y1p, src3, dst3, zeros_128)
    acc1 = a1[0, :N] + a1[1, :N]
    h = jax.nn.relu(dis[:, None] * (acc1 + y1) + b1[None, :])

    y2 = (h @ W2p) * dis[:, None]
    y2p = jnp.zeros((N_PAD, 128), jnp.float32).at[:N].set(y2)
    a2 = _agg_128(y2p, src3, dst3, zeros_128)
    acc2 = a2[0, :N] + a2[1, :N]
    out = dis[:, None] * (acc2 + y2) + b2p[None, :]
    return out[:, :40]


# speedup vs baseline: 9.9815x; 1.0086x over previous
import functools

import jax
import jax.numpy as jnp
from jax import lax
from jax.experimental import pallas as pl
from jax.experimental.pallas import tpu as pltpu
from jax.experimental.pallas import tpu_sc as plsc

N = 10000
E = 320000
N_PAD = 10240
NC = 2
NS = 16
NW = NC * NS
EPW = E // NW
CH = N_PAD // NS

B = 128
NBUF = 2
EPT = 10240
NB = EPT // B
E_PAD = NW * EPT
DUMMY_DST = N_PAD - 8

_sc_mesh = plsc.VectorSubcoreMesh(
    core_axis_name="c", subcore_axis_name="s", num_cores=NC, num_subcores=NS)


@functools.partial(
    pl.kernel,
    out_type=jax.ShapeDtypeStruct((NC, N_PAD), jnp.float32),
    mesh=_sc_mesh,
    scratch_types=[
        pltpu.VMEM((EPW,), jnp.int32),
        pltpu.VMEM((EPW,), jnp.float32),
        pltpu.VMEM_SHARED((N_PAD,), jnp.float32),
    ],
)
def _deg_kernel(dst_hbm, ones_hbm, zeros_hbm, out_hbm, idx_v, ones_v, deg_sh):
    c = lax.axis_index("c")
    s = lax.axis_index("s")
    wid = s * NC + c
    pltpu.sync_copy(zeros_hbm.at[pl.ds(s * CH, CH)], deg_sh.at[pl.ds(s * CH, CH)])
    pltpu.sync_copy(dst_hbm.at[pl.ds(wid * EPW, EPW)], idx_v)
    pltpu.sync_copy(ones_hbm, ones_v)
    plsc.subcore_barrier()
    pltpu.sync_copy(ones_v, deg_sh.at[idx_v], add=True)
    plsc.subcore_barrier()
    pltpu.sync_copy(deg_sh.at[pl.ds(s * CH, CH)], out_hbm.at[c, pl.ds(s * CH, CH)])


NCHUNK = 2
NBC = NB // NCHUNK


def _make_agg_kernel(d):
    @functools.partial(
        pl.kernel,
        out_type=jax.ShapeDtypeStruct((NC, N_PAD, d), jnp.float32),
        mesh=_sc_mesh,
        scratch_types=[
            pltpu.VMEM((NBC, B), jnp.int32),
            pltpu.VMEM((NBC, B), jnp.int32),
            [pltpu.VMEM((B, d), jnp.float32) for _ in range(NBUF)],
            pltpu.VMEM_SHARED((N_PAD, d), jnp.float32),
            [pltpu.SemaphoreType.DMA for _ in range(NBUF)],
        ],
    )
    def _agg(y_hbm, src_hbm, dst_hbm, out_hbm,
             src_v, dst_v, rows, acc_sh, sems):
        c = lax.axis_index("c")
        s = lax.axis_index("s")
        wid = s * NC + c

        z16 = jnp.zeros((16,), jnp.float32)

        def zrow(i, carry):
            for q in range(d // 16):
                rows[0][i, pl.ds(q * 16, 16)] = z16
            return carry

        lax.fori_loop(0, B, zrow, 0)

        def zbody(k, carry):
            pltpu.sync_copy(rows[0], acc_sh.at[pl.ds(s * CH + k * B, B), :])
            return carry

        lax.fori_loop(0, CH // B, zbody, 0)
        plsc.subcore_barrier()

        for c2 in range(NCHUNK):
            pltpu.sync_copy(src_hbm.at[wid, pl.ds(c2 * NBC, NBC)], src_v)
            pltpu.sync_copy(dst_hbm.at[wid, pl.ds(c2 * NBC, NBC)], dst_v)
            for k in range(NBUF):
                pltpu.async_copy(y_hbm.at[src_v.at[k]], rows[k], sems[k])

            def body(j, carry):
                for k in range(NBUF):
                    b = j * NBUF + k
                    pltpu.make_async_copy(y_hbm.at[src_v.at[b]], rows[k],
                                          sems[k]).wait()
                    pltpu.sync_copy(rows[k], acc_sh.at[dst_v.at[b]], add=True)
                    pltpu.async_copy(y_hbm.at[src_v.at[b + NBUF]], rows[k],
                                     sems[k])
                return carry

            lax.fori_loop(0, NBC // NBUF - 1, body, 0)
            for k in range(NBUF):
                b = NBC - NBUF + k
                pltpu.make_async_copy(y_hbm.at[src_v.at[b]], rows[k],
                                      sems[k]).wait()
                pltpu.sync_copy(rows[k], acc_sh.at[dst_v.at[b]], add=True)

        plsc.subcore_barrier()
        pltpu.sync_copy(acc_sh.at[pl.ds(s * CH, CH), :],
                        out_hbm.at[c, pl.ds(s * CH, CH), :])

    return _agg


_agg_128 = _make_agg_kernel(128)


BM = 256
GRID = N_PAD // BM


def _y1_body(deg_ref, x_ref, w_ref, y_ref):
    d = deg_ref[0, :] + deg_ref[1, :] + 1.0
    dis = lax.rsqrt(d)
    xw = jnp.dot(x_ref[...], w_ref[...], preferred_element_type=jnp.float32)
    y_ref[...] = xw * dis[:, None]


def _y1_call(degp, x_pad, W1):
    return pl.pallas_call(
        _y1_body,
        grid=(GRID,),
        in_specs=[
            pl.BlockSpec((2, BM), lambda i: (0, i)),
            pl.BlockSpec((BM, 128), lambda i: (i, 0)),
            pl.BlockSpec((128, 128), lambda i: (0, 0)),
        ],
        out_specs=pl.BlockSpec((BM, 128), lambda i: (i, 0)),
        out_shape=jax.ShapeDtypeStruct((N_PAD, 128), jnp.float32),
    )(degp, x_pad, W1)


def _y2_body(deg_ref, a_ref, y1_ref, b1_ref, w2_ref, y2_ref):
    d = deg_ref[0, :] + deg_ref[1, :] + 1.0
    dis = lax.rsqrt(d)
    pre = (a_ref[0] + a_ref[1] + y1_ref[...]) * dis[:, None] + b1_ref[0, :]
    h = jnp.maximum(pre, 0.0)
    hw = jnp.dot(h, w2_ref[...], preferred_element_type=jnp.float32)
    y2_ref[...] = hw * dis[:, None]


def _y2_call(degp, acc1, y1, b1, W2p):
    return pl.pallas_call(
        _y2_body,
        grid=(GRID,),
        in_specs=[
            pl.BlockSpec((2, BM), lambda i: (0, i)),
            pl.BlockSpec((2, BM, 128), lambda i: (0, i, 0)),
            pl.BlockSpec((BM, 128), lambda i: (i, 0)),
            pl.BlockSpec((1, 128), lambda i: (0, 0)),
            pl.BlockSpec((128, 128), lambda i: (0, 0)),
        ],
        out_specs=pl.BlockSpec((BM, 128), lambda i: (i, 0)),
        out_shape=jax.ShapeDtypeStruct((N_PAD, 128), jnp.float32),
    )(degp, acc1, y1, b1, W2p)


def _out_body(deg_ref, a_ref, y2_ref, b2_ref, o_ref):
    d = deg_ref[0, :] + deg_ref[1, :] + 1.0
    dis = lax.rsqrt(d)
    o_ref[...] = (a_ref[0] + a_ref[1] + y2_ref[...]) * dis[:, None] + b2_ref[0, :]


def _out_call(degp, acc2, y2, b2p):
    return pl.pallas_call(
        _out_body,
        grid=(GRID,),
        in_specs=[
            pl.BlockSpec((2, BM), lambda i: (0, i)),
            pl.BlockSpec((2, BM, 128), lambda i: (0, i, 0)),
            pl.BlockSpec((BM, 128), lambda i: (i, 0)),
            pl.BlockSpec((1, 128), lambda i: (0, 0)),
        ],
        out_specs=pl.BlockSpec((BM, 128), lambda i: (i, 0)),
        out_shape=jax.ShapeDtypeStruct((N_PAD, 128), jnp.float32),
    )(degp, acc2, y2, b2p)


def kernel(in_feat, edge_index, W1, b1, W2, b2):
    src = edge_index[0]
    dst = edge_index[1]
    ones_e = jnp.ones((EPW,), jnp.float32)
    zeros_n = jnp.zeros((N_PAD,), jnp.float32)
    degp = _deg_kernel(dst, ones_e, zeros_n)

    epad = jnp.zeros((E_PAD - E,), jnp.int32)
    src3 = jnp.concatenate([src, epad]).reshape(NW, NB, B)
    dst3 = jnp.concatenate([dst, epad + DUMMY_DST]).reshape(NW, NB, B)
    x_pad = jnp.zeros((N_PAD, 128), jnp.float32).at[:N].set(in_feat)
    W2p = jnp.zeros((128, 128), jnp.float32).at[:, :40].set(W2)
    b2p = jnp.zeros((1, 128), jnp.float32).at[0, :40].set(b2)

    y1 = _y1_call(degp, x_pad, W1)
    a1 = _agg_128(y1, src3, dst3)
    y2 = _y2_call(degp, a1, y1, b1.reshape(1, 128), W2p)
    a2 = _agg_128(y2, src3, dst3)
    out = _out_call(degp, a2, y2, b2p)
    return out[:N, :40]


# speedup vs baseline: 11.2852x; 1.1306x over previous
import functools

import jax
import jax.numpy as jnp
from jax import lax
from jax.experimental import pallas as pl
from jax.experimental.pallas import tpu as pltpu
from jax.experimental.pallas import tpu_sc as plsc

N = 10000
E = 320000
N_PAD = 10240
NC = 2
NS = 16
NW = NC * NS
EPW = E // NW
CH = N_PAD // NS

B = 128
NBUF = 2
EPT = 10240
NB = EPT // B
E_PAD = NW * EPT
DUMMY_DST = N_PAD - 8

_sc_mesh = plsc.VectorSubcoreMesh(
    core_axis_name="c", subcore_axis_name="s", num_cores=NC, num_subcores=NS)


@functools.partial(
    pl.kernel,
    out_type=jax.ShapeDtypeStruct((NC, N_PAD), jnp.float32),
    mesh=_sc_mesh,
    scratch_types=[
        pltpu.VMEM((EPW,), jnp.int32),
        pltpu.VMEM((EPW,), jnp.float32),
        pltpu.VMEM_SHARED((N_PAD,), jnp.float32),
    ],
)
def _deg_kernel(dst_hbm, ones_hbm, zeros_hbm, out_hbm, idx_v, ones_v, deg_sh):
    c = lax.axis_index("c")
    s = lax.axis_index("s")
    wid = s * NC + c
    pltpu.sync_copy(zeros_hbm.at[pl.ds(s * CH, CH)], deg_sh.at[pl.ds(s * CH, CH)])
    pltpu.sync_copy(dst_hbm.at[pl.ds(wid * EPW, EPW)], idx_v)
    pltpu.sync_copy(ones_hbm, ones_v)
    plsc.subcore_barrier()
    pltpu.sync_copy(ones_v, deg_sh.at[idx_v], add=True)
    plsc.subcore_barrier()
    pltpu.sync_copy(deg_sh.at[pl.ds(s * CH, CH)], out_hbm.at[c, pl.ds(s * CH, CH)])


NBT = 2 * NB
NB0 = 120
NB1 = NBT - NB0
CH0 = (56, 56, 8)
CH1 = (40,)
SLAB = 56


def _make_agg_kernel(d):
    @functools.partial(
        pl.kernel,
        out_type=jax.ShapeDtypeStruct((NC, N_PAD, d), jnp.float32),
        mesh=_sc_mesh,
        scratch_types=[
            pltpu.VMEM((SLAB, B), jnp.int32),
            pltpu.VMEM((SLAB, B), jnp.int32),
            [pltpu.VMEM((B, d), jnp.float32) for _ in range(NBUF)],
            pltpu.VMEM_SHARED((N_PAD, d), jnp.float32),
            [pltpu.SemaphoreType.DMA for _ in range(NBUF)],
        ],
    )
    def _agg(y_hbm, src_hbm, dst_hbm, out_hbm,
             src_v, dst_v, rows, acc_sh, sems):
        c = lax.axis_index("c")
        s = lax.axis_index("s")

        z16 = jnp.zeros((16,), jnp.float32)

        def zrow(i, carry):
            for q in range(d // 16):
                rows[0][i, pl.ds(q * 16, 16)] = z16
            return carry

        lax.fori_loop(0, B, zrow, 0)

        def zbody(k, carry):
            pltpu.sync_copy(rows[0], acc_sh.at[pl.ds(s * CH + k * B, B), :])
            return carry

        lax.fori_loop(0, CH // B, zbody, 0)
        plsc.subcore_barrier()

        def run_chunk(off, nbc):
            pltpu.sync_copy(src_hbm.at[s, pl.ds(off, nbc)],
                            src_v.at[pl.ds(0, nbc)])
            pltpu.sync_copy(dst_hbm.at[s, pl.ds(off, nbc)],
                            dst_v.at[pl.ds(0, nbc)])
            for k in range(NBUF):
                pltpu.async_copy(y_hbm.at[src_v.at[k]], rows[k], sems[k])

            def body(j, carry):
                for k in range(NBUF):
                    b = j * NBUF + k
                    pltpu.make_async_copy(y_hbm.at[src_v.at[b]], rows[k],
                                          sems[k]).wait()
                    pltpu.sync_copy(rows[k], acc_sh.at[dst_v.at[b]], add=True)
                    pltpu.async_copy(y_hbm.at[src_v.at[b + NBUF]], rows[k],
                                     sems[k])
                return carry

            lax.fori_loop(0, nbc // NBUF - 1, body, 0)
            for k in range(NBUF):
                b = nbc - NBUF + k
                pltpu.make_async_copy(y_hbm.at[src_v.at[b]], rows[k],
                                      sems[k]).wait()
                pltpu.sync_copy(rows[k], acc_sh.at[dst_v.at[b]], add=True)

        @pl.when(c == 0)
        def _():
            off = 0
            for nbc in CH0:
                run_chunk(off, nbc)
                off += nbc

        @pl.when(c == 1)
        def _():
            off = NB0
            for nbc in CH1:
                run_chunk(off, nbc)
                off += nbc

        plsc.subcore_barrier()
        pltpu.sync_copy(acc_sh.at[pl.ds(s * CH, CH), :],
                        out_hbm.at[c, pl.ds(s * CH, CH), :])

    return _agg


_agg_128 = _make_agg_kernel(128)


BM = 256
GRID = N_PAD // BM


def _y1_body(deg_ref, x_ref, w_ref, y_ref):
    d = deg_ref[0, :] + deg_ref[1, :] + 1.0
    dis = lax.rsqrt(d)
    xw = jnp.dot(x_ref[...], w_ref[...], preferred_element_type=jnp.float32)
    y_ref[...] = xw * dis[:, None]


def _y1_call(degp, x_pad, W1):
    return pl.pallas_call(
        _y1_body,
        grid=(GRID,),
        in_specs=[
            pl.BlockSpec((2, BM), lambda i: (0, i)),
            pl.BlockSpec((BM, 128), lambda i: (i, 0)),
            pl.BlockSpec((128, 128), lambda i: (0, 0)),
        ],
        out_specs=pl.BlockSpec((BM, 128), lambda i: (i, 0)),
        out_shape=jax.ShapeDtypeStruct((N_PAD, 128), jnp.float32),
    )(degp, x_pad, W1)


def _y2_body(deg_ref, a_ref, y1_ref, b1_ref, w2_ref, y2_ref):
    d = deg_ref[0, :] + deg_ref[1, :] + 1.0
    dis = lax.rsqrt(d)
    pre = (a_ref[0] + a_ref[1] + y1_ref[...]) * dis[:, None] + b1_ref[0, :]
    h = jnp.maximum(pre, 0.0)
    hw = jnp.dot(h, w2_ref[...], preferred_element_type=jnp.float32)
    y2_ref[...] = hw * dis[:, None]


def _y2_call(degp, acc1, y1, b1, W2p):
    return pl.pallas_call(
        _y2_body,
        grid=(GRID,),
        in_specs=[
            pl.BlockSpec((2, BM), lambda i: (0, i)),
            pl.BlockSpec((2, BM, 128), lambda i: (0, i, 0)),
            pl.BlockSpec((BM, 128), lambda i: (i, 0)),
            pl.BlockSpec((1, 128), lambda i: (0, 0)),
            pl.BlockSpec((128, 128), lambda i: (0, 0)),
        ],
        out_specs=pl.BlockSpec((BM, 128), lambda i: (i, 0)),
        out_shape=jax.ShapeDtypeStruct((N_PAD, 128), jnp.float32),
    )(degp, acc1, y1, b1, W2p)


def _out_body(deg_ref, a_ref, y2_ref, b2_ref, o_ref):
    d = deg_ref[0, :] + deg_ref[1, :] + 1.0
    dis = lax.rsqrt(d)
    o_ref[...] = (a_ref[0] + a_ref[1] + y2_ref[...]) * dis[:, None] + b2_ref[0, :]


def _out_call(degp, acc2, y2, b2p):
    return pl.pallas_call(
        _out_body,
        grid=(GRID,),
        in_specs=[
            pl.BlockSpec((2, BM), lambda i: (0, i)),
            pl.BlockSpec((2, BM, 128), lambda i: (0, i, 0)),
            pl.BlockSpec((BM, 128), lambda i: (i, 0)),
            pl.BlockSpec((1, 128), lambda i: (0, 0)),
        ],
        out_specs=pl.BlockSpec((BM, 128), lambda i: (i, 0)),
        out_shape=jax.ShapeDtypeStruct((N_PAD, 128), jnp.float32),
    )(degp, acc2, y2, b2p)


def kernel(in_feat, edge_index, W1, b1, W2, b2):
    src = edge_index[0]
    dst = edge_index[1]
    ones_e = jnp.ones((EPW,), jnp.float32)
    zeros_n = jnp.zeros((N_PAD,), jnp.float32)
    degp = _deg_kernel(dst, ones_e, zeros_n)

    epad = jnp.zeros((E_PAD - E,), jnp.int32)
    src3 = jnp.concatenate([src, epad]).reshape(NS, NBT, B)
    dst3 = jnp.concatenate([dst, epad + DUMMY_DST]).reshape(NS, NBT, B)
    x_pad = jnp.zeros((N_PAD, 128), jnp.float32).at[:N].set(in_feat)
    W2p = jnp.zeros((128, 128), jnp.float32).at[:, :40].set(W2)
    b2p = jnp.zeros((1, 128), jnp.float32).at[0, :40].set(b2)

    y1 = _y1_call(degp, x_pad, W1)
    a1 = _agg_128(y1, src3, dst3)
    y2 = _y2_call(degp, a1, y1, b1.reshape(1, 128), W2p)
    a2 = _agg_128(y2, src3, dst3)
    out = _out_call(degp, a2, y2, b2p)
    return out[:N, :40]


# speedup vs baseline: 20.8427x; 1.8469x over previous
import functools

import jax
import jax.numpy as jnp
from jax import lax
from jax.experimental import pallas as pl
from jax.experimental.pallas import tpu as pltpu
from jax.experimental.pallas import tpu_sc as plsc

N = 10000
E = 320000
N_PAD = 10240
NC = 2
NS = 16
NW = NC * NS
EPW = E // NW
CH = N_PAD // NS

B = 128
NBUF = 2
EPT = 10240
NB = EPT // B
E_PAD = NW * EPT
DUMMY_DST = N_PAD - 8

_sc_mesh = plsc.VectorSubcoreMesh(
    core_axis_name="c", subcore_axis_name="s", num_cores=NC, num_subcores=NS)


@functools.partial(
    pl.kernel,
    out_type=jax.ShapeDtypeStruct((NC, N_PAD), jnp.float32),
    mesh=_sc_mesh,
    scratch_types=[
        pltpu.VMEM((EPW,), jnp.int32),
        pltpu.VMEM((EPW,), jnp.float32),
        pltpu.VMEM_SHARED((N_PAD,), jnp.float32),
    ],
)
def _deg_kernel(dst_hbm, ones_hbm, zeros_hbm, out_hbm, idx_v, ones_v, deg_sh):
    c = lax.axis_index("c")
    s = lax.axis_index("s")
    wid = s * NC + c
    pltpu.sync_copy(zeros_hbm.at[pl.ds(s * CH, CH)], deg_sh.at[pl.ds(s * CH, CH)])
    pltpu.sync_copy(dst_hbm.at[pl.ds(wid * EPW, EPW)], idx_v)
    pltpu.sync_copy(ones_hbm, ones_v)
    plsc.subcore_barrier()
    pltpu.sync_copy(ones_v, deg_sh.at[idx_v], add=True)
    plsc.subcore_barrier()
    pltpu.sync_copy(deg_sh.at[pl.ds(s * CH, CH)], out_hbm.at[c, pl.ds(s * CH, CH)])


NBT = 2 * NB
DH = 64
SLAB = 80
NCHUNK = NBT // SLAB


def _make_agg_kernel():
    @functools.partial(
        pl.kernel,
        out_type=jax.ShapeDtypeStruct((NC, N_PAD, DH), jnp.float32),
        mesh=_sc_mesh,
        compiler_params=pltpu.CompilerParams(use_tc_tiling_on_sc=False),
        scratch_types=[
            pltpu.VMEM((SLAB, B), jnp.int32),
            pltpu.VMEM((SLAB, B), jnp.int32),
            [pltpu.VMEM((B, DH), jnp.float32) for _ in range(NBUF)],
            pltpu.VMEM_SHARED((N_PAD, DH), jnp.float32),
            pltpu.VMEM_SHARED((N_PAD, DH), jnp.float32),
            [pltpu.SemaphoreType.DMA for _ in range(NBUF)],
        ],
    )
    def _agg(y_hbm, src_hbm, dst_hbm, out_hbm,
             src_v, dst_v, rows, y_sh, acc_sh, sems):
        c = lax.axis_index("c")
        s = lax.axis_index("s")

        pltpu.sync_copy(y_hbm.at[c, pl.ds(s * CH, CH), :],
                        y_sh.at[pl.ds(s * CH, CH), :])

        z16 = jnp.zeros((16,), jnp.float32)

        def zrow(i, carry):
            for q in range(DH // 16):
                rows[0][i, pl.ds(q * 16, 16)] = z16
            return carry

        lax.fori_loop(0, B, zrow, 0)

        def zbody(k, carry):
            pltpu.sync_copy(rows[0], acc_sh.at[pl.ds(s * CH + k * B, B), :])
            return carry

        lax.fori_loop(0, CH // B, zbody, 0)
        plsc.subcore_barrier()

        for c2 in range(NCHUNK):
            pltpu.sync_copy(src_hbm.at[s, pl.ds(c2 * SLAB, SLAB)], src_v)
            pltpu.sync_copy(dst_hbm.at[s, pl.ds(c2 * SLAB, SLAB)], dst_v)
            for k in range(NBUF):
                pltpu.async_copy(y_sh.at[src_v.at[k]], rows[k], sems[k])

            def body(j, carry):
                for k in range(NBUF):
                    b = j * NBUF + k
                    pltpu.make_async_copy(y_sh.at[src_v.at[b]], rows[k],
                                          sems[k]).wait()
                    pltpu.sync_copy(rows[k], acc_sh.at[dst_v.at[b]], add=True)
                    pltpu.async_copy(y_sh.at[src_v.at[b + NBUF]], rows[k],
                                     sems[k])
                return carry

            lax.fori_loop(0, SLAB // NBUF - 1, body, 0)
            for k in range(NBUF):
                b = SLAB - NBUF + k
                pltpu.make_async_copy(y_sh.at[src_v.at[b]], rows[k],
                                      sems[k]).wait()
                pltpu.sync_copy(rows[k], acc_sh.at[dst_v.at[b]], add=True)

        plsc.subcore_barrier()
        pltpu.sync_copy(acc_sh.at[pl.ds(s * CH, CH), :],
                        out_hbm.at[c, pl.ds(s * CH, CH), :])

    return _agg


_agg_half = _make_agg_kernel()


BM = 256
GRID = N_PAD // BM


def _split(y):
    return jnp.stack([y[:, :DH], y[:, DH:]], axis=0)


def _y1_body(deg_ref, x_ref, w_ref, y_ref):
    d = deg_ref[0, :] + deg_ref[1, :] + 1.0
    dis = lax.rsqrt(d)
    xw = jnp.dot(x_ref[...], w_ref[...], preferred_element_type=jnp.float32)
    y_ref[...] = _split(xw * dis[:, None])


def _y1_call(degp, x_pad, W1):
    return pl.pallas_call(
        _y1_body,
        grid=(GRID,),
        in_specs=[
            pl.BlockSpec((2, BM), lambda i: (0, i)),
            pl.BlockSpec((BM, 128), lambda i: (i, 0)),
            pl.BlockSpec((128, 128), lambda i: (0, 0)),
        ],
        out_specs=pl.BlockSpec((2, BM, DH), lambda i: (0, i, 0)),
        out_shape=jax.ShapeDtypeStruct((2, N_PAD, DH), jnp.float32),
    )(degp, x_pad, W1)


def _y2_body(deg_ref, a_ref, y1_ref, b1_ref, w2_ref, y2_ref):
    d = deg_ref[0, :] + deg_ref[1, :] + 1.0
    dis = lax.rsqrt(d)
    acc = jnp.concatenate([a_ref[0] + y1_ref[0], a_ref[1] + y1_ref[1]], axis=1)
    pre = acc * dis[:, None] + b1_ref[0, :]
    h = jnp.maximum(pre, 0.0)
    hw = jnp.dot(h, w2_ref[...], preferred_element_type=jnp.float32)
    y2_ref[...] = _split(hw * dis[:, None])


def _y2_call(degp, acc1, y1, b1, W2p):
    return pl.pallas_call(
        _y2_body,
        grid=(GRID,),
        in_specs=[
            pl.BlockSpec((2, BM), lambda i: (0, i)),
            pl.BlockSpec((2, BM, DH), lambda i: (0, i, 0)),
            pl.BlockSpec((2, BM, DH), lambda i: (0, i, 0)),
            pl.BlockSpec((1, 128), lambda i: (0, 0)),
            pl.BlockSpec((128, 128), lambda i: (0, 0)),
        ],
        out_specs=pl.BlockSpec((2, BM, DH), lambda i: (0, i, 0)),
        out_shape=jax.ShapeDtypeStruct((2, N_PAD, DH), jnp.float32),
    )(degp, acc1, y1, b1, W2p)


def _out_body(deg_ref, a_ref, y2_ref, b2_ref, o_ref):
    d = deg_ref[0, :] + deg_ref[1, :] + 1.0
    dis = lax.rsqrt(d)
    acc = jnp.concatenate([a_ref[0] + y2_ref[0], a_ref[1] + y2_ref[1]], axis=1)
    o_ref[...] = acc * dis[:, None] + b2_ref[0, :]


def _out_call(degp, acc2, y2, b2p):
    return pl.pallas_call(
        _out_body,
        grid=(GRID,),
        in_specs=[
            pl.BlockSpec((2, BM), lambda i: (0, i)),
            pl.BlockSpec((2, BM, DH), lambda i: (0, i, 0)),
            pl.BlockSpec((2, BM, DH), lambda i: (0, i, 0)),
            pl.BlockSpec((1, 128), lambda i: (0, 0)),
        ],
        out_specs=pl.BlockSpec((BM, 128), lambda i: (i, 0)),
        out_shape=jax.ShapeDtypeStruct((N_PAD, 128), jnp.float32),
    )(degp, acc2, y2, b2p)


def kernel(in_feat, edge_index, W1, b1, W2, b2):
    src = edge_index[0]
    dst = edge_index[1]
    ones_e = jnp.ones((EPW,), jnp.float32)
    zeros_n = jnp.zeros((N_PAD,), jnp.float32)
    degp = _deg_kernel(dst, ones_e, zeros_n)

    epad = jnp.zeros((E_PAD - E,), jnp.int32)
    src3 = jnp.concatenate([src, epad]).reshape(NS, NBT, B)
    dst3 = jnp.concatenate([dst, epad + DUMMY_DST]).reshape(NS, NBT, B)
    x_pad = jnp.zeros((N_PAD, 128), jnp.float32).at[:N].set(in_feat)
    W2p = jnp.zeros((128, 128), jnp.float32).at[:, :40].set(W2)
    b2p = jnp.zeros((1, 128), jnp.float32).at[0, :40].set(b2)

    y1 = _y1_call(degp, x_pad, W1)
    a1 = _agg_half(y1, src3, dst3)
    y2 = _y2_call(degp, a1, y1, b1.reshape(1, 128), W2p)
    a2 = _agg_half(y2, src3, dst3)
    out = _out_call(degp, a2, y2, b2p)
    return out[:N, :40]


# speedup vs baseline: 29.3898x; 1.4101x over previous
import functools

import jax
import jax.numpy as jnp
from jax import lax
from jax.experimental import pallas as pl
from jax.experimental.pallas import tpu as pltpu
from jax.experimental.pallas import tpu_sc as plsc

N = 10000
E = 320000
N_PAD = 10240
NC = 2
NS = 16
NW = NC * NS
EPW = E // NW
CH = N_PAD // NS

B = 128
NBUF = 2
EPT = 10240
NB = EPT // B
E_PAD = NW * EPT
DUMMY_DST = N_PAD - 8

_sc_mesh = plsc.VectorSubcoreMesh(
    core_axis_name="c", subcore_axis_name="s", num_cores=NC, num_subcores=NS)


@functools.partial(
    pl.kernel,
    out_type=jax.ShapeDtypeStruct((NC, N_PAD), jnp.float32),
    mesh=_sc_mesh,
    scratch_types=[
        pltpu.VMEM((EPW,), jnp.int32),
        pltpu.VMEM((EPW,), jnp.float32),
        pltpu.VMEM_SHARED((N_PAD,), jnp.float32),
    ],
)
def _deg_kernel(dst_hbm, ones_hbm, zeros_hbm, out_hbm, idx_v, ones_v, deg_sh):
    c = lax.axis_index("c")
    s = lax.axis_index("s")
    wid = s * NC + c
    pltpu.sync_copy(zeros_hbm.at[pl.ds(s * CH, CH)], deg_sh.at[pl.ds(s * CH, CH)])
    pltpu.sync_copy(dst_hbm.at[pl.ds(wid * EPW, EPW)], idx_v)
    pltpu.sync_copy(ones_hbm, ones_v)
    plsc.subcore_barrier()
    pltpu.sync_copy(ones_v, deg_sh.at[idx_v], add=True)
    plsc.subcore_barrier()
    pltpu.sync_copy(deg_sh.at[pl.ds(s * CH, CH)], out_hbm.at[c, pl.ds(s * CH, CH)])


NBT = 2 * NB
DH = 64
SLAB = 80
NCHUNK = NBT // SLAB


AGG_DT = jnp.bfloat16


def _make_agg_kernel():
    @functools.partial(
        pl.kernel,
        out_type=jax.ShapeDtypeStruct((NC, N_PAD, DH), AGG_DT),
        mesh=_sc_mesh,
        compiler_params=pltpu.CompilerParams(use_tc_tiling_on_sc=False),
        scratch_types=[
            pltpu.VMEM((SLAB, B), jnp.int32),
            pltpu.VMEM((SLAB, B), jnp.int32),
            [pltpu.VMEM((B, DH), AGG_DT) for _ in range(NBUF)],
            pltpu.VMEM_SHARED((N_PAD, DH), AGG_DT),
            pltpu.VMEM_SHARED((N_PAD, DH), AGG_DT),
            [pltpu.SemaphoreType.DMA for _ in range(NBUF)],
        ],
    )
    def _agg(y_hbm, src_hbm, dst_hbm, out_hbm,
             src_v, dst_v, rows, y_sh, acc_sh, sems):
        c = lax.axis_index("c")
        s = lax.axis_index("s")

        pltpu.sync_copy(y_hbm.at[c, pl.ds(s * CH, CH), :],
                        y_sh.at[pl.ds(s * CH, CH), :])

        zlanes = 16 * 4 // jnp.dtype(AGG_DT).itemsize
        zv = jnp.zeros((zlanes,), AGG_DT)

        def zrow(i, carry):
            for q in range(DH // zlanes):
                rows[0][i, pl.ds(q * zlanes, zlanes)] = zv
            return carry

        lax.fori_loop(0, B, zrow, 0)

        def zbody(k, carry):
            pltpu.sync_copy(rows[0], acc_sh.at[pl.ds(s * CH + k * B, B), :])
            return carry

        lax.fori_loop(0, CH // B, zbody, 0)
        plsc.subcore_barrier()

        for c2 in range(NCHUNK):
            pltpu.sync_copy(src_hbm.at[s, pl.ds(c2 * SLAB, SLAB)], src_v)
            pltpu.sync_copy(dst_hbm.at[s, pl.ds(c2 * SLAB, SLAB)], dst_v)
            for k in range(NBUF):
                pltpu.async_copy(y_sh.at[src_v.at[k]], rows[k], sems[k])

            def body(j, carry):
                for k in range(NBUF):
                    b = j * NBUF + k
                    pltpu.make_async_copy(y_sh.at[src_v.at[b]], rows[k],
                                          sems[k]).wait()
                    pltpu.sync_copy(rows[k], acc_sh.at[dst_v.at[b]], add=True)
                    pltpu.async_copy(y_sh.at[src_v.at[b + NBUF]], rows[k],
                                     sems[k])
                return carry

            lax.fori_loop(0, SLAB // NBUF - 1, body, 0)
            for k in range(NBUF):
                b = SLAB - NBUF + k
                pltpu.make_async_copy(y_sh.at[src_v.at[b]], rows[k],
                                      sems[k]).wait()
                pltpu.sync_copy(rows[k], acc_sh.at[dst_v.at[b]], add=True)

        plsc.subcore_barrier()
        pltpu.sync_copy(acc_sh.at[pl.ds(s * CH, CH), :],
                        out_hbm.at[c, pl.ds(s * CH, CH), :])

    return _agg


_agg_half = _make_agg_kernel()


BM = 256
GRID = N_PAD // BM


def _split(y):
    return jnp.stack([y[:, :DH], y[:, DH:]], axis=0).astype(AGG_DT)


def _y1_body(deg_ref, x_ref, w_ref, y_ref):
    d = deg_ref[0, :] + deg_ref[1, :] + 1.0
    dis = lax.rsqrt(d)
    xw = jnp.dot(x_ref[...], w_ref[...], preferred_element_type=jnp.float32)
    y_ref[...] = _split(xw * dis[:, None])


def _y1_call(degp, x_pad, W1):
    return pl.pallas_call(
        _y1_body,
        grid=(GRID,),
        in_specs=[
            pl.BlockSpec((2, BM), lambda i: (0, i)),
            pl.BlockSpec((BM, 128), lambda i: (i, 0)),
            pl.BlockSpec((128, 128), lambda i: (0, 0)),
        ],
        out_specs=pl.BlockSpec((2, BM, DH), lambda i: (0, i, 0)),
        out_shape=jax.ShapeDtypeStruct((2, N_PAD, DH), AGG_DT),
    )(degp, x_pad, W1)


def _y2_body(deg_ref, a_ref, y1_ref, b1_ref, w2_ref, y2_ref):
    d = deg_ref[0, :] + deg_ref[1, :] + 1.0
    dis = lax.rsqrt(d)
    acc = jnp.concatenate(
        [a_ref[0].astype(jnp.float32) + y1_ref[0].astype(jnp.float32),
         a_ref[1].astype(jnp.float32) + y1_ref[1].astype(jnp.float32)], axis=1)
    pre = acc * dis[:, None] + b1_ref[0, :]
    h = jnp.maximum(pre, 0.0)
    hw = jnp.dot(h, w2_ref[...], preferred_element_type=jnp.float32)
    y2_ref[...] = _split(hw * dis[:, None])


def _y2_call(degp, acc1, y1, b1, W2p):
    return pl.pallas_call(
        _y2_body,
        grid=(GRID,),
        in_specs=[
            pl.BlockSpec((2, BM), lambda i: (0, i)),
            pl.BlockSpec((2, BM, DH), lambda i: (0, i, 0)),
            pl.BlockSpec((2, BM, DH), lambda i: (0, i, 0)),
            pl.BlockSpec((1, 128), lambda i: (0, 0)),
            pl.BlockSpec((128, 128), lambda i: (0, 0)),
        ],
        out_specs=pl.BlockSpec((2, BM, DH), lambda i: (0, i, 0)),
        out_shape=jax.ShapeDtypeStruct((2, N_PAD, DH), AGG_DT),
    )(degp, acc1, y1, b1, W2p)


def _out_body(deg_ref, a_ref, y2_ref, b2_ref, o_ref):
    d = deg_ref[0, :] + deg_ref[1, :] + 1.0
    dis = lax.rsqrt(d)
    acc = jnp.concatenate(
        [a_ref[0].astype(jnp.float32) + y2_ref[0].astype(jnp.float32),
         a_ref[1].astype(jnp.float32) + y2_ref[1].astype(jnp.float32)], axis=1)
    o_ref[...] = acc * dis[:, None] + b2_ref[0, :]


def _out_call(degp, acc2, y2, b2p):
    return pl.pallas_call(
        _out_body,
        grid=(GRID,),
        in_specs=[
            pl.BlockSpec((2, BM), lambda i: (0, i)),
            pl.BlockSpec((2, BM, DH), lambda i: (0, i, 0)),
            pl.BlockSpec((2, BM, DH), lambda i: (0, i, 0)),
            pl.BlockSpec((1, 128), lambda i: (0, 0)),
        ],
        out_specs=pl.BlockSpec((BM, 128), lambda i: (i, 0)),
        out_shape=jax.ShapeDtypeStruct((N_PAD, 128), jnp.float32),
    )(degp, acc2, y2, b2p)


def kernel(in_feat, edge_index, W1, b1, W2, b2):
    src = edge_index[0]
    dst = edge_index[1]
    ones_e = jnp.ones((EPW,), jnp.float32)
    zeros_n = jnp.zeros((N_PAD,), jnp.float32)
    degp = _deg_kernel(dst, ones_e, zeros_n)

    epad = jnp.zeros((E_PAD - E,), jnp.int32)
    src3 = jnp.concatenate([src, epad]).reshape(NS, NBT, B)
    dst3 = jnp.concatenate([dst, epad + DUMMY_DST]).reshape(NS, NBT, B)
    x_pad = jnp.zeros((N_PAD, 128), jnp.float32).at[:N].set(in_feat)
    W2p = jnp.zeros((128, 128), jnp.float32).at[:, :40].set(W2)
    b2p = jnp.zeros((1, 128), jnp.float32).at[0, :40].set(b2)

    y1 = _y1_call(degp, x_pad, W1)
    a1 = _agg_half(y1, src3, dst3)
    y2 = _y2_call(degp, a1, y1, b1.reshape(1, 128), W2p)
    a2 = _agg_half(y2, src3, dst3)
    out = _out_call(degp, a2, y2, b2p)
    return out[:N, :40]


# speedup vs baseline: 33.1320x; 1.1273x over previous
import functools

import jax
import jax.numpy as jnp
from jax import lax
from jax.experimental import pallas as pl
from jax.experimental.pallas import tpu as pltpu
from jax.experimental.pallas import tpu_sc as plsc

N = 10000
E = 320000
N_PAD = 10240
NC = 2
NS = 16
NW = NC * NS
EPW = E // NW
CH = N_PAD // NS

B = 128
NBUF = 4
EPT = 10240
NB = EPT // B
E_PAD = NW * EPT
DUMMY_DST = N_PAD - 8

_sc_mesh = plsc.VectorSubcoreMesh(
    core_axis_name="c", subcore_axis_name="s", num_cores=NC, num_subcores=NS)


@functools.partial(
    pl.kernel,
    out_type=jax.ShapeDtypeStruct((NC, N_PAD), jnp.float32),
    mesh=_sc_mesh,
    scratch_types=[
        pltpu.VMEM((EPW,), jnp.int32),
        pltpu.VMEM((EPW,), jnp.float32),
        pltpu.VMEM_SHARED((N_PAD,), jnp.float32),
    ],
)
def _deg_kernel(dst_hbm, ones_hbm, zeros_hbm, out_hbm, idx_v, ones_v, deg_sh):
    c = lax.axis_index("c")
    s = lax.axis_index("s")
    wid = s * NC + c
    pltpu.sync_copy(zeros_hbm.at[pl.ds(s * CH, CH)], deg_sh.at[pl.ds(s * CH, CH)])
    pltpu.sync_copy(dst_hbm.at[pl.ds(wid * EPW, EPW)], idx_v)
    pltpu.sync_copy(ones_hbm, ones_v)
    plsc.subcore_barrier()
    pltpu.sync_copy(ones_v, deg_sh.at[idx_v], add=True)
    plsc.subcore_barrier()
    pltpu.sync_copy(deg_sh.at[pl.ds(s * CH, CH)], out_hbm.at[c, pl.ds(s * CH, CH)])


NBT = 2 * NB
DH = 64
SLAB = 160
NCHUNK = NBT // SLAB


AGG_DT = jnp.bfloat16


def _make_agg_kernel():
    @functools.partial(
        pl.kernel,
        out_type=jax.ShapeDtypeStruct((NC, N_PAD, DH), AGG_DT),
        mesh=_sc_mesh,
        compiler_params=pltpu.CompilerParams(use_tc_tiling_on_sc=False),
        scratch_types=[
            pltpu.VMEM((SLAB, B), jnp.int32),
            pltpu.VMEM((SLAB, B), jnp.int32),
            [pltpu.VMEM((B, DH), AGG_DT) for _ in range(NBUF)],
            pltpu.VMEM_SHARED((N_PAD, DH), AGG_DT),
            pltpu.VMEM_SHARED((N_PAD, DH), AGG_DT),
            [pltpu.SemaphoreType.DMA for _ in range(NBUF)],
        ],
    )
    def _agg(y_hbm, src_hbm, dst_hbm, out_hbm,
             src_v, dst_v, rows, y_sh, acc_sh, sems):
        c = lax.axis_index("c")
        s = lax.axis_index("s")

        pltpu.sync_copy(y_hbm.at[c, pl.ds(s * CH, CH), :],
                        y_sh.at[pl.ds(s * CH, CH), :])

        zlanes = 16 * 4 // jnp.dtype(AGG_DT).itemsize
        zv = jnp.zeros((zlanes,), AGG_DT)

        def zrow(i, carry):
            for q in range(DH // zlanes):
                rows[0][i, pl.ds(q * zlanes, zlanes)] = zv
            return carry

        lax.fori_loop(0, B, zrow, 0)

        def zbody(k, carry):
            pltpu.sync_copy(rows[0], acc_sh.at[pl.ds(s * CH + k * B, B), :])
            return carry

        lax.fori_loop(0, CH // B, zbody, 0)
        plsc.subcore_barrier()

        for c2 in range(NCHUNK):
            pltpu.sync_copy(src_hbm.at[s, pl.ds(c2 * SLAB, SLAB)], src_v)
            pltpu.sync_copy(dst_hbm.at[s, pl.ds(c2 * SLAB, SLAB)], dst_v)
            for k in range(NBUF):
                pltpu.async_copy(y_sh.at[src_v.at[k]], rows[k], sems[k])

            def body(j, carry):
                for k in range(NBUF):
                    b = j * NBUF + k
                    pltpu.make_async_copy(y_sh.at[src_v.at[b]], rows[k],
                                          sems[k]).wait()
                    pltpu.sync_copy(rows[k], acc_sh.at[dst_v.at[b]], add=True)
                    pltpu.async_copy(y_sh.at[src_v.at[b + NBUF]], rows[k],
                                     sems[k])
                return carry

            lax.fori_loop(0, SLAB // NBUF - 1, body, 0)
            for k in range(NBUF):
                b = SLAB - NBUF + k
                pltpu.make_async_copy(y_sh.at[src_v.at[b]], rows[k],
                                      sems[k]).wait()
                pltpu.sync_copy(rows[k], acc_sh.at[dst_v.at[b]], add=True)

        plsc.subcore_barrier()
        pltpu.sync_copy(acc_sh.at[pl.ds(s * CH, CH), :],
                        out_hbm.at[c, pl.ds(s * CH, CH), :])

    return _agg


_agg_half = _make_agg_kernel()


BM = 512
GRID = N_PAD // BM


def _split(y):
    return jnp.stack([y[:, :DH], y[:, DH:]], axis=0).astype(AGG_DT)


def _y1_body(deg_ref, x_ref, w_ref, y_ref):
    d = deg_ref[0, :] + deg_ref[1, :] + 1.0
    dis = lax.rsqrt(d)
    xw = jnp.dot(x_ref[...], w_ref[...], preferred_element_type=jnp.float32)
    y_ref[...] = _split(xw * dis[:, None])


def _y1_call(degp, x_pad, W1):
    return pl.pallas_call(
        _y1_body,
        grid=(GRID,),
        in_specs=[
            pl.BlockSpec((2, BM), lambda i: (0, i)),
            pl.BlockSpec((BM, 128), lambda i: (i, 0)),
            pl.BlockSpec((128, 128), lambda i: (0, 0)),
        ],
        out_specs=pl.BlockSpec((2, BM, DH), lambda i: (0, i, 0)),
        out_shape=jax.ShapeDtypeStruct((2, N_PAD, DH), AGG_DT),
    )(degp, x_pad, W1)


def _y2_body(deg_ref, a_ref, y1_ref, b1_ref, w2_ref, y2_ref):
    d = deg_ref[0, :] + deg_ref[1, :] + 1.0
    dis = lax.rsqrt(d)
    acc = jnp.concatenate(
        [a_ref[0].astype(jnp.float32) + y1_ref[0].astype(jnp.float32),
         a_ref[1].astype(jnp.float32) + y1_ref[1].astype(jnp.float32)], axis=1)
    pre = acc * dis[:, None] + b1_ref[0, :]
    h = jnp.maximum(pre, 0.0)
    hw = jnp.dot(h, w2_ref[...], preferred_element_type=jnp.float32)
    y2_ref[...] = _split(hw * dis[:, None])


def _y2_call(degp, acc1, y1, b1, W2p):
    return pl.pallas_call(
        _y2_body,
        grid=(GRID,),
        in_specs=[
            pl.BlockSpec((2, BM), lambda i: (0, i)),
            pl.BlockSpec((2, BM, DH), lambda i: (0, i, 0)),
            pl.BlockSpec((2, BM, DH), lambda i: (0, i, 0)),
            pl.BlockSpec((1, 128), lambda i: (0, 0)),
            pl.BlockSpec((128, 128), lambda i: (0, 0)),
        ],
        out_specs=pl.BlockSpec((2, BM, DH), lambda i: (0, i, 0)),
        out_shape=jax.ShapeDtypeStruct((2, N_PAD, DH), AGG_DT),
    )(degp, acc1, y1, b1, W2p)


def _out_body(deg_ref, a_ref, y2_ref, b2_ref, o_ref):
    d = deg_ref[0, :] + deg_ref[1, :] + 1.0
    dis = lax.rsqrt(d)
    acc = jnp.concatenate(
        [a_ref[0].astype(jnp.float32) + y2_ref[0].astype(jnp.float32),
         a_ref[1].astype(jnp.float32) + y2_ref[1].astype(jnp.float32)], axis=1)
    o_ref[...] = acc * dis[:, None] + b2_ref[0, :]


def _out_call(degp, acc2, y2, b2p):
    return pl.pallas_call(
        _out_body,
        grid=(GRID,),
        in_specs=[
            pl.BlockSpec((2, BM), lambda i: (0, i)),
            pl.BlockSpec((2, BM, DH), lambda i: (0, i, 0)),
            pl.BlockSpec((2, BM, DH), lambda i: (0, i, 0)),
            pl.BlockSpec((1, 128), lambda i: (0, 0)),
        ],
        out_specs=pl.BlockSpec((BM, 128), lambda i: (i, 0)),
        out_shape=jax.ShapeDtypeStruct((N_PAD, 128), jnp.float32),
    )(degp, acc2, y2, b2p)


def kernel(in_feat, edge_index, W1, b1, W2, b2):
    src = edge_index[0]
    dst = edge_index[1]
    ones_e = jnp.ones((EPW,), jnp.float32)
    zeros_n = jnp.zeros((N_PAD,), jnp.float32)
    degp = _deg_kernel(dst, ones_e, zeros_n)

    epad = jnp.zeros((E_PAD - E,), jnp.int32)
    src3 = jnp.concatenate([src, epad]).reshape(NS, NBT, B)
    dst3 = jnp.concatenate([dst, epad + DUMMY_DST]).reshape(NS, NBT, B)
    x_pad = jnp.zeros((N_PAD, 128), jnp.float32).at[:N].set(in_feat)
    W2p = jnp.zeros((128, 128), jnp.float32).at[:, :40].set(W2)
    b2p = jnp.zeros((1, 128), jnp.float32).at[0, :40].set(b2)

    y1 = _y1_call(degp, x_pad, W1)
    a1 = _agg_half(y1, src3, dst3)
    y2 = _y2_call(degp, a1, y1, b1.reshape(1, 128), W2p)
    a2 = _agg_half(y2, src3, dst3)
    out = _out_call(degp, a2, y2, b2p)
    return out[:N, :40]


# speedup vs baseline: 34.7837x; 1.0499x over previous
import functools

import jax
import jax.numpy as jnp
from jax import lax
from jax.experimental import pallas as pl
from jax.experimental.pallas import tpu as pltpu
from jax.experimental.pallas import tpu_sc as plsc

N = 10000
E = 320000
N_PAD = 10240
NC = 2
NS = 16
NW = NC * NS
EPW = E // NW
CH = N_PAD // NS

B = 128
NBUF = 8
EPT = 10240
NB = EPT // B
E_PAD = NW * EPT
DUMMY_DST = N_PAD - 8

_sc_mesh = plsc.VectorSubcoreMesh(
    core_axis_name="c", subcore_axis_name="s", num_cores=NC, num_subcores=NS)


@functools.partial(
    pl.kernel,
    out_type=jax.ShapeDtypeStruct((NC, N_PAD), jnp.float32),
    mesh=_sc_mesh,
    scratch_types=[
        pltpu.VMEM((EPW,), jnp.int32),
        pltpu.VMEM((EPW,), jnp.float32),
        pltpu.VMEM_SHARED((N_PAD,), jnp.float32),
    ],
)
def _deg_kernel(dst_hbm, ones_hbm, zeros_hbm, out_hbm, idx_v, ones_v, deg_sh):
    c = lax.axis_index("c")
    s = lax.axis_index("s")
    wid = s * NC + c
    pltpu.sync_copy(zeros_hbm.at[pl.ds(s * CH, CH)], deg_sh.at[pl.ds(s * CH, CH)])
    pltpu.sync_copy(dst_hbm.at[pl.ds(wid * EPW, EPW)], idx_v)
    pltpu.sync_copy(ones_hbm, ones_v)
    plsc.subcore_barrier()
    pltpu.sync_copy(ones_v, deg_sh.at[idx_v], add=True)
    plsc.subcore_barrier()
    pltpu.sync_copy(deg_sh.at[pl.ds(s * CH, CH)], out_hbm.at[c, pl.ds(s * CH, CH)])


NBT = 2 * NB
DH = 64
SLAB = 160
NCHUNK = NBT // SLAB


AGG_DT = jnp.bfloat16


def _make_agg_kernel():
    @functools.partial(
        pl.kernel,
        out_type=jax.ShapeDtypeStruct((NC, N_PAD, DH), AGG_DT),
        mesh=_sc_mesh,
        compiler_params=pltpu.CompilerParams(use_tc_tiling_on_sc=False),
        scratch_types=[
            pltpu.VMEM((SLAB, B), jnp.int32),
            pltpu.VMEM((SLAB, B), jnp.int32),
            [pltpu.VMEM((B, DH), AGG_DT) for _ in range(NBUF)],
            pltpu.VMEM_SHARED((N_PAD, DH), AGG_DT),
            pltpu.VMEM_SHARED((N_PAD, DH), AGG_DT),
            [pltpu.SemaphoreType.DMA for _ in range(NBUF)],
        ],
    )
    def _agg(y_hbm, src_hbm, dst_hbm, out_hbm,
             src_v, dst_v, rows, y_sh, acc_sh, sems):
        c = lax.axis_index("c")
        s = lax.axis_index("s")

        pltpu.sync_copy(y_hbm.at[c, pl.ds(s * CH, CH), :],
                        y_sh.at[pl.ds(s * CH, CH), :])

        zlanes = 16 * 4 // jnp.dtype(AGG_DT).itemsize
        zv = jnp.zeros((zlanes,), AGG_DT)

        def zrow(i, carry):
            for q in range(DH // zlanes):
                rows[0][i, pl.ds(q * zlanes, zlanes)] = zv
            return carry

        lax.fori_loop(0, B, zrow, 0)

        def zbody(k, carry):
            pltpu.sync_copy(rows[0], acc_sh.at[pl.ds(s * CH + k * B, B), :])
            return carry

        lax.fori_loop(0, CH // B, zbody, 0)
        plsc.subcore_barrier()

        for c2 in range(NCHUNK):
            pltpu.sync_copy(src_hbm.at[s, pl.ds(c2 * SLAB, SLAB)], src_v)
            pltpu.sync_copy(dst_hbm.at[s, pl.ds(c2 * SLAB, SLAB)], dst_v)
            for k in range(NBUF):
                pltpu.async_copy(y_sh.at[src_v.at[k]], rows[k], sems[k])

            def body(j, carry):
                for k in range(NBUF):
                    b = j * NBUF + k
                    pltpu.make_async_copy(y_sh.at[src_v.at[b]], rows[k],
                                          sems[k]).wait()
                    pltpu.sync_copy(rows[k], acc_sh.at[dst_v.at[b]], add=True)
                    pltpu.async_copy(y_sh.at[src_v.at[b + NBUF]], rows[k],
                                     sems[k])
                return carry

            lax.fori_loop(0, SLAB // NBUF - 1, body, 0)
            for k in range(NBUF):
                b = SLAB - NBUF + k
                pltpu.make_async_copy(y_sh.at[src_v.at[b]], rows[k],
                                      sems[k]).wait()
                pltpu.sync_copy(rows[k], acc_sh.at[dst_v.at[b]], add=True)

        plsc.subcore_barrier()
        pltpu.sync_copy(acc_sh.at[pl.ds(s * CH, CH), :],
                        out_hbm.at[c, pl.ds(s * CH, CH), :])

    return _agg


_agg_half = _make_agg_kernel()


BM = 1024
GRID = N_PAD // BM


def _split(y):
    return jnp.stack([y[:, :DH], y[:, DH:]], axis=0).astype(AGG_DT)


def _y1_body(deg_ref, x_ref, w_ref, y_ref):
    d = deg_ref[0, :] + deg_ref[1, :] + 1.0
    dis = lax.rsqrt(d)
    xw = jnp.dot(x_ref[...], w_ref[...], preferred_element_type=jnp.float32)
    y_ref[...] = _split(xw * dis[:, None])


def _y1_call(degp, x_pad, W1):
    return pl.pallas_call(
        _y1_body,
        grid=(GRID,),
        in_specs=[
            pl.BlockSpec((2, BM), lambda i: (0, i)),
            pl.BlockSpec((BM, 128), lambda i: (i, 0)),
            pl.BlockSpec((128, 128), lambda i: (0, 0)),
        ],
        out_specs=pl.BlockSpec((2, BM, DH), lambda i: (0, i, 0)),
        out_shape=jax.ShapeDtypeStruct((2, N_PAD, DH), AGG_DT),
    )(degp, x_pad, W1)


def _y2_body(deg_ref, a_ref, y1_ref, b1_ref, w2_ref, y2_ref):
    d = deg_ref[0, :] + deg_ref[1, :] + 1.0
    dis = lax.rsqrt(d)
    acc = jnp.concatenate(
        [a_ref[0].astype(jnp.float32) + y1_ref[0].astype(jnp.float32),
         a_ref[1].astype(jnp.float32) + y1_ref[1].astype(jnp.float32)], axis=1)
    pre = acc * dis[:, None] + b1_ref[0, :]
    h = jnp.maximum(pre, 0.0)
    hw = jnp.dot(h, w2_ref[...], preferred_element_type=jnp.float32)
    y2_ref[...] = _split(hw * dis[:, None])


def _y2_call(degp, acc1, y1, b1, W2p):
    return pl.pallas_call(
        _y2_body,
        grid=(GRID,),
        in_specs=[
            pl.BlockSpec((2, BM), lambda i: (0, i)),
            pl.BlockSpec((2, BM, DH), lambda i: (0, i, 0)),
            pl.BlockSpec((2, BM, DH), lambda i: (0, i, 0)),
            pl.BlockSpec((1, 128), lambda i: (0, 0)),
            pl.BlockSpec((128, 128), lambda i: (0, 0)),
        ],
        out_specs=pl.BlockSpec((2, BM, DH), lambda i: (0, i, 0)),
        out_shape=jax.ShapeDtypeStruct((2, N_PAD, DH), AGG_DT),
    )(degp, acc1, y1, b1, W2p)


def _out_body(deg_ref, a_ref, y2_ref, b2_ref, o_ref):
    d = deg_ref[0, :] + deg_ref[1, :] + 1.0
    dis = lax.rsqrt(d)
    acc = jnp.concatenate(
        [a_ref[0].astype(jnp.float32) + y2_ref[0].astype(jnp.float32),
         a_ref[1].astype(jnp.float32) + y2_ref[1].astype(jnp.float32)], axis=1)
    o_ref[...] = acc * dis[:, None] + b2_ref[0, :]


def _out_call(degp, acc2, y2, b2p):
    return pl.pallas_call(
        _out_body,
        grid=(GRID,),
        in_specs=[
            pl.BlockSpec((2, BM), lambda i: (0, i)),
            pl.BlockSpec((2, BM, DH), lambda i: (0, i, 0)),
            pl.BlockSpec((2, BM, DH), lambda i: (0, i, 0)),
            pl.BlockSpec((1, 128), lambda i: (0, 0)),
        ],
        out_specs=pl.BlockSpec((BM, 128), lambda i: (i, 0)),
        out_shape=jax.ShapeDtypeStruct((N_PAD, 128), jnp.float32),
    )(degp, acc2, y2, b2p)


def kernel(in_feat, edge_index, W1, b1, W2, b2):
    src = edge_index[0]
    dst = edge_index[1]
    ones_e = jnp.ones((EPW,), jnp.float32)
    zeros_n = jnp.zeros((N_PAD,), jnp.float32)
    degp = _deg_kernel(dst, ones_e, zeros_n)

    epad = jnp.zeros((E_PAD - E,), jnp.int32)
    src3 = jnp.concatenate([src, epad]).reshape(NS, NBT, B)
    dst3 = jnp.concatenate([dst, epad + DUMMY_DST]).reshape(NS, NBT, B)
    x_pad = jnp.zeros((N_PAD, 128), jnp.float32).at[:N].set(in_feat)
    W2p = jnp.zeros((128, 128), jnp.float32).at[:, :40].set(W2)
    b2p = jnp.zeros((1, 128), jnp.float32).at[0, :40].set(b2)

    y1 = _y1_call(degp, x_pad, W1)
    a1 = _agg_half(y1, src3, dst3)
    y2 = _y2_call(degp, a1, y1, b1.reshape(1, 128), W2p)
    a2 = _agg_half(y2, src3, dst3)
    out = _out_call(degp, a2, y2, b2p)
    return out[:N, :40]
